# baseline (device time: 55075 ns/iter reference)
import os
import jax
import jax.numpy as jnp
from jax import lax
from jax.experimental import pallas as pl
from jax.experimental.pallas import tpu as pltpu

N_DEV = 16
HQ_PER = 8
DH = 128
SQ = 256
SKV = 4096
DM = 1024
CH = SQ // N_DEV
HALF = SQ // 2
SCALE = 0.08838834764831843
BF = jnp.bfloat16

_NO_AR = os.environ.get('NOBFLY') == '1'


def _body(x_ref, wq_ref, k_ref, v_ref, wo_ref, out_ref,
          q_ref, bias_ref, ctx_ref, kall, vall, rbuf,
          kvsem, ssems1, ssems2, rsems1, rsems2):
    my = lax.axis_index("i")

    def kv_copies(h):
        gh = my * HQ_PER + h
        return (
            pltpu.make_async_copy(k_ref.at[0, :, gh, :],
                                  kall.at[:, pl.ds(h * DH, DH)], kvsem.at[0, h]),
            pltpu.make_async_copy(v_ref.at[0, :, gh, :],
                                  vall.at[:, pl.ds(h * DH, DH)], kvsem.at[1, h]),
        )

    for h in range(HQ_PER):
        kcp, vcp = kv_copies(h)
        kcp.start()
        vcp.start()

    def p1_rdma(c):
        return pltpu.make_async_remote_copy(
            src_ref=out_ref.at[pl.ds(c * CH, CH), :],
            dst_ref=rbuf.at[my],
            send_sem=ssems1.at[c],
            recv_sem=rsems1.at[my],
            device_id=(c,),
            device_id_type=pl.DeviceIdType.MESH,
        )

    q_ref[...] = jnp.dot(x_ref[...].astype(BF), wq_ref[...].astype(BF),
                         preferred_element_type=jnp.float32)

    qb = lax.broadcasted_iota(jnp.int32, (SQ, SKV), 0) // 64
    kb = lax.broadcasted_iota(jnp.int32, (SQ, SKV), 1) // 64
    mask = (qb == kb) | (kb == 0) | (lax.rem(qb + kb, 3) == 0)
    bias_ref[...] = jnp.where(mask, 0.0, -1e9)

    for h in range(HQ_PER):
        kcp, vcp = kv_copies(h)
        kcp.wait()
        vcp.wait()
        qh = q_ref[:, h * DH:(h + 1) * DH].astype(BF)
        kh = kall[:, h * DH:(h + 1) * DH].astype(BF)
        s = lax.dot_general(qh, kh, (((1,), (1,)), ((), ())),
                            preferred_element_type=jnp.float32)
        w = jnp.exp(s * SCALE + bias_ref[...])
        r = 1.0 / jnp.sum(w, axis=-1, keepdims=True)
        ctx_ref[:, h * DH:(h + 1) * DH] = jnp.dot(
            w.astype(BF), vall[:, h * DH:(h + 1) * DH].astype(BF),
            preferred_element_type=jnp.float32) * r

    out_ref[...] = jnp.dot(ctx_ref[...].astype(BF), wo_ref[...].astype(BF),
                           preferred_element_type=jnp.float32)

    if _NO_AR:
        return

    barrier = pltpu.get_barrier_semaphore()
    for d in range(1, N_DEV):
        pl.semaphore_signal(barrier, inc=1,
                            device_id=(lax.rem(my + d, N_DEV),),
                            device_id_type=pl.DeviceIdType.MESH)
    pl.semaphore_wait(barrier, N_DEV - 1)

    for c in range(N_DEV):
        @pl.when(my != c)
        def _(c=c):
            p1_rdma(c).start()

    own = my * CH
    acc = out_ref[pl.ds(own, CH), :]
    for d in range(1, N_DEV):
        src = lax.rem(my - d + N_DEV, N_DEV)
        recv = pltpu.make_async_remote_copy(
            src_ref=rbuf.at[src], dst_ref=rbuf.at[src],
            send_sem=ssems1.at[0], recv_sem=rsems1.at[src],
            device_id=(src,), device_id_type=pl.DeviceIdType.MESH,
        )
        recv.wait_recv()
        acc = acc + rbuf[src]
    out_ref[pl.ds(own, CH), :] = acc

    p2 = []
    for d in range(1, N_DEV):
        dst = lax.rem(my + d, N_DEV)
        rdma = pltpu.make_async_remote_copy(
            src_ref=out_ref.at[pl.ds(own, CH), :],
            dst_ref=out_ref.at[pl.ds(own, CH), :],
            send_sem=ssems2.at[d - 1],
            recv_sem=rsems2.at[my],
            device_id=(dst,),
            device_id_type=pl.DeviceIdType.MESH,
        )
        rdma.start()
        p2.append(rdma)

    for d in range(1, N_DEV):
        src = lax.rem(my - d + N_DEV, N_DEV)
        recv = pltpu.make_async_remote_copy(
            src_ref=out_ref.at[pl.ds(src * CH, CH), :],
            dst_ref=out_ref.at[pl.ds(src * CH, CH), :],
            send_sem=ssems2.at[0], recv_sem=rsems2.at[src],
            device_id=(src,), device_id_type=pl.DeviceIdType.MESH,
        )
        recv.wait_recv()

    for c in range(N_DEV):
        @pl.when(my != c)
        def _(c=c):
            p1_rdma(c).wait_send()
    for rdma in p2:
        rdma.wait_send()


def kernel(x, Wq, K_ext, V_ext, Wo):
    x2 = x.reshape(SQ, DM)

    out = pl.pallas_call(
        _body,
        out_shape=jax.ShapeDtypeStruct((SQ, DM), jnp.float32),
        in_specs=[
            pl.BlockSpec(memory_space=pltpu.VMEM),
            pl.BlockSpec(memory_space=pltpu.VMEM),
            pl.BlockSpec(memory_space=pltpu.MemorySpace.HBM),
            pl.BlockSpec(memory_space=pltpu.MemorySpace.HBM),
            pl.BlockSpec(memory_space=pltpu.VMEM),
        ],
        out_specs=pl.BlockSpec(memory_space=pltpu.VMEM),
        scratch_shapes=[
            pltpu.VMEM((SQ, DM), jnp.float32),
            pltpu.VMEM((SQ, SKV), jnp.float32),
            pltpu.VMEM((SQ, DM), jnp.float32),
            pltpu.VMEM((SKV, HQ_PER * DH), jnp.float32),
            pltpu.VMEM((SKV, HQ_PER * DH), jnp.float32),
            pltpu.VMEM((N_DEV, CH, DM), jnp.float32),
            pltpu.SemaphoreType.DMA((2, HQ_PER)),
            pltpu.SemaphoreType.DMA((N_DEV,)),
            pltpu.SemaphoreType.DMA((N_DEV - 1,)),
            pltpu.SemaphoreType.DMA((N_DEV,)),
            pltpu.SemaphoreType.DMA((N_DEV,)),
        ],
        compiler_params=(pltpu.CompilerParams(vmem_limit_bytes=60 * 1024 * 1024)
                         if _NO_AR else
                         pltpu.CompilerParams(
                             collective_id=0, vmem_limit_bytes=60 * 1024 * 1024)),
    )(x2, Wq, K_ext, V_ext, Wo)
    return out.reshape(1, SQ, DM)


# device time: 46734 ns/iter; 1.1785x vs baseline; 1.1785x over previous
import os
import jax
import jax.numpy as jnp
from jax import lax
from jax.experimental import pallas as pl
from jax.experimental.pallas import tpu as pltpu

N_DEV = 16
HQ_PER = 8
DH = 128
SQ = 256
SKV = 4096
DM = 1024
CH = SQ // N_DEV
HALF = SQ // 2
SCALE = 0.08838834764831843
BF = jnp.bfloat16

_NO_AR = os.environ.get('NOBFLY') == '1'


def _body(x_ref, wq_ref, k_ref, v_ref, wo_ref, out_ref,
          q_ref, bias_ref, ctx_ref, kall, vall, rbuf, pbuf,
          kvsem, ssems1, ssems2, rsems1, rsems2):
    my = lax.axis_index("i")

    def kv_copies(h):
        gh = my * HQ_PER + h
        return (
            pltpu.make_async_copy(k_ref.at[0, :, gh, :],
                                  kall.at[:, pl.ds(h * DH, DH)], kvsem.at[0, h]),
            pltpu.make_async_copy(v_ref.at[0, :, gh, :],
                                  vall.at[:, pl.ds(h * DH, DH)], kvsem.at[1, h]),
        )

    for h in range(HQ_PER):
        kcp, vcp = kv_copies(h)
        kcp.start()
        vcp.start()

    def p1_rdma(c):
        return pltpu.make_async_remote_copy(
            src_ref=pbuf.at[pl.ds(c * CH, CH), :],
            dst_ref=rbuf.at[my],
            send_sem=ssems1.at[c],
            recv_sem=rsems1.at[my],
            device_id=(c,),
            device_id_type=pl.DeviceIdType.MESH,
        )

    q_ref[...] = jnp.dot(x_ref[...].astype(BF), wq_ref[...].astype(BF),
                         preferred_element_type=jnp.float32)

    qb = lax.broadcasted_iota(jnp.int32, (SQ, SKV), 0) // 64
    kb = lax.broadcasted_iota(jnp.int32, (SQ, SKV), 1) // 64
    mask = (qb == kb) | (kb == 0) | (lax.rem(qb + kb, 3) == 0)
    bias_ref[...] = jnp.where(mask, 0.0, -1e9)

    for h in range(HQ_PER):
        kcp, vcp = kv_copies(h)
        kcp.wait()
        vcp.wait()
        qh = q_ref[:, h * DH:(h + 1) * DH].astype(BF)
        kh = kall[:, h * DH:(h + 1) * DH].astype(BF)
        s = lax.dot_general(qh, kh, (((1,), (1,)), ((), ())),
                            preferred_element_type=jnp.float32)
        w = jnp.exp(s * SCALE + bias_ref[...])
        r = 1.0 / jnp.sum(w, axis=-1, keepdims=True)
        ctx_ref[:, h * DH:(h + 1) * DH] = jnp.dot(
            w.astype(BF), vall[:, h * DH:(h + 1) * DH].astype(BF),
            preferred_element_type=jnp.float32) * r

    partial = jnp.dot(ctx_ref[...].astype(BF), wo_ref[...].astype(BF),
                      preferred_element_type=jnp.float32)
    out_ref[...] = partial
    pbuf[...] = partial.astype(BF)

    if _NO_AR:
        return

    barrier = pltpu.get_barrier_semaphore()
    for d in range(1, N_DEV):
        pl.semaphore_signal(barrier, inc=1,
                            device_id=(lax.rem(my + d, N_DEV),),
                            device_id_type=pl.DeviceIdType.MESH)
    pl.semaphore_wait(barrier, N_DEV - 1)

    for c in range(N_DEV):
        @pl.when(my != c)
        def _(c=c):
            p1_rdma(c).start()

    own = my * CH
    acc = out_ref[pl.ds(own, CH), :]
    for d in range(1, N_DEV):
        src = lax.rem(my - d + N_DEV, N_DEV)
        recv = pltpu.make_async_remote_copy(
            src_ref=rbuf.at[src], dst_ref=rbuf.at[src],
            send_sem=ssems1.at[0], recv_sem=rsems1.at[src],
            device_id=(src,), device_id_type=pl.DeviceIdType.MESH,
        )
        recv.wait_recv()
        acc = acc + rbuf[src].astype(jnp.float32)
    pbuf[pl.ds(own, CH), :] = acc.astype(BF)

    p2 = []
    for d in range(1, N_DEV):
        dst = lax.rem(my + d, N_DEV)
        rdma = pltpu.make_async_remote_copy(
            src_ref=pbuf.at[pl.ds(own, CH), :],
            dst_ref=pbuf.at[pl.ds(own, CH), :],
            send_sem=ssems2.at[d - 1],
            recv_sem=rsems2.at[my],
            device_id=(dst,),
            device_id_type=pl.DeviceIdType.MESH,
        )
        rdma.start()
        p2.append(rdma)

    for d in range(1, N_DEV):
        src = lax.rem(my - d + N_DEV, N_DEV)
        recv = pltpu.make_async_remote_copy(
            src_ref=pbuf.at[pl.ds(src * CH, CH), :],
            dst_ref=pbuf.at[pl.ds(src * CH, CH), :],
            send_sem=ssems2.at[0], recv_sem=rsems2.at[src],
            device_id=(src,), device_id_type=pl.DeviceIdType.MESH,
        )
        recv.wait_recv()

    out_ref[...] = pbuf[...].astype(jnp.float32)
    out_ref[pl.ds(own, CH), :] = acc

    for c in range(N_DEV):
        @pl.when(my != c)
        def _(c=c):
            p1_rdma(c).wait_send()
    for rdma in p2:
        rdma.wait_send()


def kernel(x, Wq, K_ext, V_ext, Wo):
    x2 = x.reshape(SQ, DM)

    out = pl.pallas_call(
        _body,
        out_shape=jax.ShapeDtypeStruct((SQ, DM), jnp.float32),
        in_specs=[
            pl.BlockSpec(memory_space=pltpu.VMEM),
            pl.BlockSpec(memory_space=pltpu.VMEM),
            pl.BlockSpec(memory_space=pltpu.MemorySpace.HBM),
            pl.BlockSpec(memory_space=pltpu.MemorySpace.HBM),
            pl.BlockSpec(memory_space=pltpu.VMEM),
        ],
        out_specs=pl.BlockSpec(memory_space=pltpu.VMEM),
        scratch_shapes=[
            pltpu.VMEM((SQ, DM), jnp.float32),
            pltpu.VMEM((SQ, SKV), jnp.float32),
            pltpu.VMEM((SQ, DM), jnp.float32),
            pltpu.VMEM((SKV, HQ_PER * DH), jnp.float32),
            pltpu.VMEM((SKV, HQ_PER * DH), jnp.float32),
            pltpu.VMEM((N_DEV, CH, DM), BF),
            pltpu.VMEM((SQ, DM), BF),
            pltpu.SemaphoreType.DMA((2, HQ_PER)),
            pltpu.SemaphoreType.DMA((N_DEV,)),
            pltpu.SemaphoreType.DMA((N_DEV - 1,)),
            pltpu.SemaphoreType.DMA((N_DEV,)),
            pltpu.SemaphoreType.DMA((N_DEV,)),
        ],
        compiler_params=(pltpu.CompilerParams(vmem_limit_bytes=60 * 1024 * 1024)
                         if _NO_AR else
                         pltpu.CompilerParams(
                             collective_id=0, vmem_limit_bytes=60 * 1024 * 1024)),
    )(x2, Wq, K_ext, V_ext, Wo)
    return out.reshape(1, SQ, DM)


# device time: 42326 ns/iter; 1.3012x vs baseline; 1.1041x over previous
import os
import jax
import jax.numpy as jnp
from jax import lax
from jax.experimental import pallas as pl
from jax.experimental.pallas import tpu as pltpu

N_DEV = 16
HQ_PER = 8
DH = 128
SQ = 256
SKV = 4096
DM = 1024
CH = SQ // N_DEV
HALF = SQ // 2
SCALE = 0.08838834764831843
BF = jnp.bfloat16

_NO_AR = os.environ.get('NOBFLY') == '1'


def _body(x_ref, wq_ref, k_ref, v_ref, wo_ref, out_ref,
          q_ref, bias_ref, ctx_ref, kall, vall, rbuf, pbuf,
          kvsem, ssems1, ssems2, rsems1, rsems2):
    my = lax.axis_index("i")

    def kv_copies(h):
        gh = my * HQ_PER + h
        return (
            pltpu.make_async_copy(k_ref.at[0, :, gh, :],
                                  kall.at[:, pl.ds(h * DH, DH)], kvsem.at[0, h]),
            pltpu.make_async_copy(v_ref.at[0, :, gh, :],
                                  vall.at[:, pl.ds(h * DH, DH)], kvsem.at[1, h]),
        )

    for h in range(HQ_PER):
        kcp, vcp = kv_copies(h)
        kcp.start()
        vcp.start()

    def p1_rdma(c):
        return pltpu.make_async_remote_copy(
            src_ref=pbuf.at[pl.ds(c * CH, CH), :],
            dst_ref=rbuf.at[my],
            send_sem=ssems1.at[c],
            recv_sem=rsems1.at[my],
            device_id=(c,),
            device_id_type=pl.DeviceIdType.MESH,
        )

    q_ref[...] = jnp.dot(x_ref[...].astype(BF), wq_ref[...].astype(BF),
                         preferred_element_type=jnp.float32) * SCALE

    qb = lax.broadcasted_iota(jnp.int32, (SQ, SKV), 0) // 64
    kb = lax.broadcasted_iota(jnp.int32, (SQ, SKV), 1) // 64
    mask = (qb == kb) | (kb == 0) | (lax.rem(qb + kb, 3) == 0)
    bias_ref[...] = jnp.where(mask, 0.0, -1e9).astype(BF)

    for h in range(HQ_PER):
        kcp, vcp = kv_copies(h)
        kcp.wait()
        vcp.wait()
        qh = q_ref[:, h * DH:(h + 1) * DH].astype(BF)
        kh = kall[:, h * DH:(h + 1) * DH].astype(BF)
        s = lax.dot_general(qh, kh, (((1,), (1,)), ((), ())),
                            preferred_element_type=jnp.float32)
        w = jnp.exp(s.astype(BF) + bias_ref[...])
        r = 1.0 / jnp.sum(w, axis=-1, keepdims=True, dtype=jnp.float32)
        ctx_ref[:, h * DH:(h + 1) * DH] = jnp.dot(
            w, vall[:, h * DH:(h + 1) * DH].astype(BF),
            preferred_element_type=jnp.float32) * r

    partial = jnp.dot(ctx_ref[...].astype(BF), wo_ref[...].astype(BF),
                      preferred_element_type=jnp.float32)
    out_ref[...] = partial
    pbuf[...] = partial.astype(BF)

    if _NO_AR:
        return

    barrier = pltpu.get_barrier_semaphore()
    for d in range(1, N_DEV):
        pl.semaphore_signal(barrier, inc=1,
                            device_id=(lax.rem(my + d, N_DEV),),
                            device_id_type=pl.DeviceIdType.MESH)
    pl.semaphore_wait(barrier, N_DEV - 1)

    for c in range(N_DEV):
        @pl.when(my != c)
        def _(c=c):
            p1_rdma(c).start()

    own = my * CH
    acc = out_ref[pl.ds(own, CH), :]
    for d in range(1, N_DEV):
        src = lax.rem(my - d + N_DEV, N_DEV)
        recv = pltpu.make_async_remote_copy(
            src_ref=rbuf.at[src], dst_ref=rbuf.at[src],
            send_sem=ssems1.at[0], recv_sem=rsems1.at[src],
            device_id=(src,), device_id_type=pl.DeviceIdType.MESH,
        )
        recv.wait_recv()
        acc = acc + rbuf[src].astype(jnp.float32)
    pbuf[pl.ds(own, CH), :] = acc.astype(BF)

    p2 = []
    for d in range(1, N_DEV):
        dst = lax.rem(my + d, N_DEV)
        rdma = pltpu.make_async_remote_copy(
            src_ref=pbuf.at[pl.ds(own, CH), :],
            dst_ref=pbuf.at[pl.ds(own, CH), :],
            send_sem=ssems2.at[d - 1],
            recv_sem=rsems2.at[my],
            device_id=(dst,),
            device_id_type=pl.DeviceIdType.MESH,
        )
        rdma.start()
        p2.append(rdma)

    for d in range(1, N_DEV):
        src = lax.rem(my - d + N_DEV, N_DEV)
        recv = pltpu.make_async_remote_copy(
            src_ref=pbuf.at[pl.ds(src * CH, CH), :],
            dst_ref=pbuf.at[pl.ds(src * CH, CH), :],
            send_sem=ssems2.at[0], recv_sem=rsems2.at[src],
            device_id=(src,), device_id_type=pl.DeviceIdType.MESH,
        )
        recv.wait_recv()

    out_ref[...] = pbuf[...].astype(jnp.float32)
    out_ref[pl.ds(own, CH), :] = acc

    for c in range(N_DEV):
        @pl.when(my != c)
        def _(c=c):
            p1_rdma(c).wait_send()
    for rdma in p2:
        rdma.wait_send()


def kernel(x, Wq, K_ext, V_ext, Wo):
    x2 = x.reshape(SQ, DM)

    out = pl.pallas_call(
        _body,
        out_shape=jax.ShapeDtypeStruct((SQ, DM), jnp.float32),
        in_specs=[
            pl.BlockSpec(memory_space=pltpu.VMEM),
            pl.BlockSpec(memory_space=pltpu.VMEM),
            pl.BlockSpec(memory_space=pltpu.MemorySpace.HBM),
            pl.BlockSpec(memory_space=pltpu.MemorySpace.HBM),
            pl.BlockSpec(memory_space=pltpu.VMEM),
        ],
        out_specs=pl.BlockSpec(memory_space=pltpu.VMEM),
        scratch_shapes=[
            pltpu.VMEM((SQ, DM), jnp.float32),
            pltpu.VMEM((SQ, SKV), BF),
            pltpu.VMEM((SQ, DM), jnp.float32),
            pltpu.VMEM((SKV, HQ_PER * DH), jnp.float32),
            pltpu.VMEM((SKV, HQ_PER * DH), jnp.float32),
            pltpu.VMEM((N_DEV, CH, DM), BF),
            pltpu.VMEM((SQ, DM), BF),
            pltpu.SemaphoreType.DMA((2, HQ_PER)),
            pltpu.SemaphoreType.DMA((N_DEV,)),
            pltpu.SemaphoreType.DMA((N_DEV - 1,)),
            pltpu.SemaphoreType.DMA((N_DEV,)),
            pltpu.SemaphoreType.DMA((N_DEV,)),
        ],
        compiler_params=(pltpu.CompilerParams(vmem_limit_bytes=60 * 1024 * 1024)
                         if _NO_AR else
                         pltpu.CompilerParams(
                             collective_id=0, vmem_limit_bytes=60 * 1024 * 1024)),
    )(x2, Wq, K_ext, V_ext, Wo)
    return out.reshape(1, SQ, DM)
